# Initial kernel scaffold; baseline (speedup 1.0000x reference)
import functools

import jax
import jax.numpy as jnp
from jax import lax
from jax.experimental import pallas as pl
from jax.experimental.pallas import tpu as pltpu
from jax.experimental.pallas import tpu_sc as plsc

NC = 2
NS = 16
NW = NC * NS
L = 16
EC = 128


def _sc_mesh():
    return plsc.VectorSubcoreMesh(core_axis_name="c", subcore_axis_name="s",
                                  num_cores=NC, num_subcores=NS)


def _deg_kernel(n, et):
    @functools.partial(
        pl.kernel,
        out_type=jax.ShapeDtypeStruct((NW, n), jnp.float32),
        mesh=_sc_mesh(),
        scratch_types=[
            pltpu.VMEM((et,), jnp.int32),
            pltpu.VMEM((et,), jnp.float32),
            pltpu.VMEM((n,), jnp.float32),
        ],
    )
    def k(dst_hbm, ew_hbm, degp_hbm, dst_v, ew_v, part_v):
        w = lax.axis_index("s") * NC + lax.axis_index("c")
        z = jnp.zeros((L,), jnp.float32)

        def zbody(i, carry):
            part_v[pl.ds(i * L, L)] = z
            return carry

        lax.fori_loop(0, n // L, zbody, 0)
        pltpu.sync_copy(dst_hbm.at[w], dst_v)
        pltpu.sync_copy(ew_hbm.at[w], ew_v)

        def body(j, carry):
            idx = dst_v[pl.ds(j * L, L)]
            vals = ew_v[pl.ds(j * L, L)]
            plsc.addupdate_scatter(part_v, [idx], vals)
            return carry

        lax.fori_loop(0, et // L, body, 0)
        pltpu.sync_copy(part_v, degp_hbm.at[w])

    return k


def _norm_kernel(n, et):
    @functools.partial(
        pl.kernel,
        out_type=jax.ShapeDtypeStruct((NW, et), jnp.float32),
        mesh=_sc_mesh(),
        scratch_types=[
            pltpu.VMEM((n,), jnp.float32),
            pltpu.VMEM((et,), jnp.int32),
            pltpu.VMEM((et,), jnp.int32),
            pltpu.VMEM((et,), jnp.float32),
            pltpu.VMEM((et,), jnp.float32),
        ],
    )
    def k(src_hbm, dst_hbm, ew_hbm, dinv_hbm, norm_hbm,
          dinv_v, src_v, dst_v, ew_v, out_v):
        w = lax.axis_index("s") * NC + lax.axis_index("c")
        pltpu.sync_copy(dinv_hbm, dinv_v)
        pltpu.sync_copy(src_hbm.at[w], src_v)
        pltpu.sync_copy(dst_hbm.at[w], dst_v)
        pltpu.sync_copy(ew_hbm.at[w], ew_v)

        def body(j, carry):
            s = plsc.load_gather(dinv_v, [src_v[pl.ds(j * L, L)]])
            t = plsc.load_gather(dinv_v, [dst_v[pl.ds(j * L, L)]])
            out_v[pl.ds(j * L, L)] = s * ew_v[pl.ds(j * L, L)] * t
            return carry

        lax.fori_loop(0, et // L, body, 0)
        pltpu.sync_copy(out_v, norm_hbm.at[w])

    return k


def _agg_kernel(n, d, ch):
    ns_rows = n // NS

    @functools.partial(
        pl.kernel,
        out_type=jax.ShapeDtypeStruct((NC, n, d), jnp.float32),
        mesh=_sc_mesh(),
        scratch_types=[
            pltpu.VMEM((ch, EC), jnp.int32),
            pltpu.VMEM((ch, EC), jnp.int32),
            pltpu.VMEM((ch, EC), jnp.float32),
            pltpu.VMEM((EC, d), jnp.float32),
            pltpu.VMEM_SHARED((n, d), jnp.float32),
            pltpu.SemaphoreType.DMA,
        ],
    )
    def k(g_hbm, src_hbm, dst_hbm, norm_hbm, init_hbm, zero_hbm, aggp_hbm,
          src_v, dst_v, norm_v, rows_v, agg_sp, sem):
        c = lax.axis_index("c")
        s = lax.axis_index("s")
        w = s * NC + c
        r0 = s * ns_rows

        @pl.when(c == 0)
        def _():
            pltpu.sync_copy(init_hbm.at[pl.ds(r0, ns_rows)],
                            agg_sp.at[pl.ds(r0, ns_rows)])

        @pl.when(c != 0)
        def _():
            pltpu.sync_copy(zero_hbm.at[pl.ds(r0, ns_rows)],
                            agg_sp.at[pl.ds(r0, ns_rows)])

        pltpu.sync_copy(src_hbm.at[w], src_v)
        pltpu.sync_copy(dst_hbm.at[w], dst_v)
        pltpu.sync_copy(norm_hbm.at[w], norm_v)
        plsc.subcore_barrier()

        def chunk(j, carry):
            pltpu.async_copy(g_hbm.at[src_v.at[j]], rows_v, sem).wait()

            def scale(r, carry2):
                sv = jnp.full((L,), norm_v[j, r], jnp.float32)
                for kk in range(d // L):
                    rows_v[r, pl.ds(kk * L, L)] = (
                        rows_v[r, pl.ds(kk * L, L)] * sv)
                return carry2

            lax.fori_loop(0, EC, scale, 0)
            pltpu.sync_copy(rows_v, agg_sp.at[dst_v.at[j]], add=True)
            return carry

        lax.fori_loop(0, ch, chunk, 0)
        plsc.subcore_barrier()
        pltpu.sync_copy(agg_sp.at[pl.ds(r0, ns_rows)],
                        aggp_hbm.at[c, pl.ds(r0, ns_rows)])

    return k


def _prep_tc(x, w1, degp):
    n, _ = x.shape
    dh = w1.shape[1]

    def body(x_ref, w_ref, degp_ref, h_ref, self_ref, dinv_ref):
        deg = jnp.sum(degp_ref[...], axis=0) + 1.0
        dinv = lax.rsqrt(deg)
        h = jnp.dot(x_ref[...], w_ref[...],
                    preferred_element_type=jnp.float32)
        h_ref[...] = h
        self_ref[...] = h * (dinv * dinv)[:, None]
        dinv_ref[...] = dinv

    return pl.pallas_call(
        body,
        out_shape=(
            jax.ShapeDtypeStruct((n, dh), jnp.float32),
            jax.ShapeDtypeStruct((n, dh), jnp.float32),
            jax.ShapeDtypeStruct((n,), jnp.float32),
        ),
    )(x, w1, degp)


def _mid_tc(aggp, b1, w2, dinv):
    _, n, dh = aggp.shape
    dl = w2.shape[1]

    def body(aggp_ref, b_ref, w_ref, dinv_ref, h2_ref, self2_ref):
        z = jnp.maximum(aggp_ref[0] + aggp_ref[1] + b_ref[...], 0.0)
        h2 = jnp.dot(z, w_ref[...], preferred_element_type=jnp.float32)
        dinv = dinv_ref[...]
        h2_ref[...] = h2
        self2_ref[...] = h2 * (dinv * dinv)[:, None]

    return pl.pallas_call(
        body,
        out_shape=(
            jax.ShapeDtypeStruct((n, dl), jnp.float32),
            jax.ShapeDtypeStruct((n, dl), jnp.float32),
        ),
    )(aggp, b1, w2, dinv)


def _fin_tc(aggp, b2):
    _, n, dl = aggp.shape

    def body(aggp_ref, b_ref, out_ref):
        out_ref[...] = jnp.maximum(aggp_ref[0] + aggp_ref[1] + b_ref[...],
                                   0.0)

    return pl.pallas_call(
        body,
        out_shape=jax.ShapeDtypeStruct((n, dl), jnp.float32),
    )(aggp, b2)


def kernel(x, edge_index, edge_weight, W1, b1, W2, b2):
    n, _ = x.shape
    e = edge_weight.shape[0]
    dh = W1.shape[1]
    dl = W2.shape[1]

    src = edge_index[0]
    dst = edge_index[1]
    per_tile = -(-e // NW)
    et = -(-per_tile // EC) * EC
    epad = NW * et
    padn = epad - e
    src_p = jnp.concatenate(
        [src, jnp.zeros((padn,), src.dtype)]).reshape(NW, et)
    dst_p = jnp.concatenate(
        [dst, jnp.zeros((padn,), dst.dtype)]).reshape(NW, et)
    ew_p = jnp.concatenate(
        [edge_weight, jnp.zeros((padn,), edge_weight.dtype)]).reshape(NW, et)

    degp = _deg_kernel(n, et)(dst_p, ew_p)
    h1, self1, dinv = _prep_tc(x, W1, degp)
    norm2 = _norm_kernel(n, et)(src_p, dst_p, ew_p, dinv)

    ch = et // EC
    src3 = src_p.reshape(NW, ch, EC)
    dst3 = dst_p.reshape(NW, ch, EC)
    norm3 = norm2.reshape(NW, ch, EC)

    zeros1 = jnp.zeros((n, dh), jnp.float32)
    aggp1 = _agg_kernel(n, dh, ch)(h1, src3, dst3, norm3, self1, zeros1)
    h2, self2 = _mid_tc(aggp1, b1, W2, dinv)

    zeros2 = jnp.zeros((n, dl), jnp.float32)
    aggp2 = _agg_kernel(n, dl, ch)(h2, src3, dst3, norm3, self2, zeros2)
    return _fin_tc(aggp2, b2)


# speedup vs baseline: 11.0673x; 11.0673x over previous
import functools

import jax
import jax.numpy as jnp
from jax import lax
from jax.experimental import pallas as pl
from jax.experimental.pallas import tpu as pltpu
from jax.experimental.pallas import tpu_sc as plsc

NC = 2
NS = 16
NW = NC * NS
L = 16
EC = 128


def _sc_mesh():
    return plsc.VectorSubcoreMesh(core_axis_name="c", subcore_axis_name="s",
                                  num_cores=NC, num_subcores=NS)


def _deg_kernel(n, npad, et):
    @functools.partial(
        pl.kernel,
        out_type=jax.ShapeDtypeStruct((NW, npad), jnp.float32),
        mesh=_sc_mesh(),
        scratch_types=[
            pltpu.VMEM((et,), jnp.int32),
            pltpu.VMEM((et,), jnp.float32),
            pltpu.VMEM((npad,), jnp.float32),
        ],
        compiler_params=pltpu.CompilerParams(needs_layout_passes=False),
    )
    def k(dst_hbm, ew_hbm, degp_hbm, dst_v, ew_v, part_v):
        w = lax.axis_index("s") * NC + lax.axis_index("c")
        z = jnp.zeros((L,), jnp.float32)

        def zbody(i, carry):
            part_v[pl.ds(i * L, L)] = z
            return carry

        lax.fori_loop(0, npad // L, zbody, 0)
        pltpu.sync_copy(dst_hbm.at[w], dst_v)
        pltpu.sync_copy(ew_hbm.at[w], ew_v)

        def body(j, carry):
            idx = dst_v[pl.ds(j * L, L)]
            vals = ew_v[pl.ds(j * L, L)]
            plsc.addupdate_scatter(part_v, [idx], vals)
            return carry

        lax.fori_loop(0, et // L, body, 0)
        pltpu.sync_copy(part_v, degp_hbm.at[w])

    return k


def _norm_kernel(npad, et):
    @functools.partial(
        pl.kernel,
        out_type=jax.ShapeDtypeStruct((NW, et), jnp.float32),
        mesh=_sc_mesh(),
        scratch_types=[
            pltpu.VMEM((npad,), jnp.float32),
            pltpu.VMEM((et,), jnp.int32),
            pltpu.VMEM((et,), jnp.int32),
            pltpu.VMEM((et,), jnp.float32),
            pltpu.VMEM((et,), jnp.float32),
        ],
        compiler_params=pltpu.CompilerParams(needs_layout_passes=False),
    )
    def k(src_hbm, dst_hbm, ew_hbm, dinv_hbm, norm_hbm,
          dinv_v, src_v, dst_v, ew_v, out_v):
        w = lax.axis_index("s") * NC + lax.axis_index("c")
        pltpu.sync_copy(dinv_hbm, dinv_v)
        pltpu.sync_copy(src_hbm.at[w], src_v)
        pltpu.sync_copy(dst_hbm.at[w], dst_v)
        pltpu.sync_copy(ew_hbm.at[w], ew_v)

        def body(j, carry):
            s = plsc.load_gather(dinv_v, [src_v[pl.ds(j * L, L)]])
            t = plsc.load_gather(dinv_v, [dst_v[pl.ds(j * L, L)]])
            out_v[pl.ds(j * L, L)] = s * ew_v[pl.ds(j * L, L)] * t
            return carry

        lax.fori_loop(0, et // L, body, 0)
        pltpu.sync_copy(out_v, norm_hbm.at[w])

    return k


def _agg_kernel(n, d, ch):
    ns_rows = n // NS

    @functools.partial(
        pl.kernel,
        out_type=jax.ShapeDtypeStruct((NC, n, d), jnp.float32),
        mesh=_sc_mesh(),
        scratch_types=[
            pltpu.VMEM((ch, EC), jnp.int32),
            pltpu.VMEM((ch, EC), jnp.int32),
            pltpu.VMEM((ch, EC), jnp.float32),
            pltpu.VMEM((EC, d), jnp.float32),
            pltpu.VMEM_SHARED((n, d), jnp.float32),
            pltpu.SemaphoreType.DMA,
        ],
        compiler_params=pltpu.CompilerParams(needs_layout_passes=False,
                                             use_tc_tiling_on_sc=False),
    )
    def k(g_hbm, src_hbm, dst_hbm, norm_hbm, init_hbm, zero_hbm, aggp_hbm,
          src_v, dst_v, norm_v, rows_v, agg_sp, sem):
        c = lax.axis_index("c")
        s = lax.axis_index("s")
        w = s * NC + c
        r0 = s * ns_rows

        @pl.when(c == 0)
        def _():
            pltpu.sync_copy(init_hbm.at[pl.ds(r0, ns_rows)],
                            agg_sp.at[pl.ds(r0, ns_rows)])

        @pl.when(c != 0)
        def _():
            pltpu.sync_copy(zero_hbm.at[pl.ds(r0, ns_rows)],
                            agg_sp.at[pl.ds(r0, ns_rows)])

        pltpu.sync_copy(src_hbm.at[w], src_v)
        pltpu.sync_copy(dst_hbm.at[w], dst_v)
        pltpu.sync_copy(norm_hbm.at[w], norm_v)
        plsc.subcore_barrier()

        def chunk(j, carry):
            pltpu.async_copy(g_hbm.at[src_v.at[j]], rows_v, sem).wait()

            def scale(g, carry2):
                nv = norm_v[j, pl.ds(g * L, L)]
                for rr in range(L):
                    sv = jnp.full((L,), nv[rr], jnp.float32)
                    r = g * L + rr
                    for kk in range(d // L):
                        rows_v[r, pl.ds(kk * L, L)] = (
                            rows_v[r, pl.ds(kk * L, L)] * sv)
                return carry2

            lax.fori_loop(0, EC // L, scale, 0)
            pltpu.sync_copy(rows_v, agg_sp.at[dst_v.at[j]], add=True)
            return carry

        lax.fori_loop(0, ch, chunk, 0)
        plsc.subcore_barrier()
        pltpu.sync_copy(agg_sp.at[pl.ds(r0, ns_rows)],
                        aggp_hbm.at[c, pl.ds(r0, ns_rows)])

    return k


def _prep_tc(x, w1, degp):
    n, _ = x.shape
    dh = w1.shape[1]

    def body(x_ref, w_ref, degp_ref, h_ref, self_ref, dinv_ref):
        deg = jnp.sum(degp_ref[...], axis=0) + 1.0
        dinv = lax.rsqrt(deg)
        h = jnp.dot(x_ref[...], w_ref[...],
                    preferred_element_type=jnp.float32)
        h_ref[...] = h
        self_ref[...] = h * (dinv * dinv)[:, None]
        dinv_ref[...] = dinv

    return pl.pallas_call(
        body,
        out_shape=(
            jax.ShapeDtypeStruct((n, dh), jnp.float32),
            jax.ShapeDtypeStruct((n, dh), jnp.float32),
            jax.ShapeDtypeStruct((n,), jnp.float32),
        ),
    )(x, w1, degp)


def _mid_tc(aggp, b1, w2, dinv):
    _, n, dh = aggp.shape
    dl = w2.shape[1]

    def body(aggp_ref, b_ref, w_ref, dinv_ref, h2_ref, self2_ref):
        z = jnp.maximum(aggp_ref[0] + aggp_ref[1] + b_ref[...], 0.0)
        h2 = jnp.dot(z, w_ref[...], preferred_element_type=jnp.float32)
        dinv = dinv_ref[...]
        h2_ref[...] = h2
        self2_ref[...] = h2 * (dinv * dinv)[:, None]

    return pl.pallas_call(
        body,
        out_shape=(
            jax.ShapeDtypeStruct((n, dl), jnp.float32),
            jax.ShapeDtypeStruct((n, dl), jnp.float32),
        ),
    )(aggp, b1, w2, dinv)


def _fin_tc(aggp, b2, n):
    _, npad, dl = aggp.shape

    def body(aggp_ref, b_ref, out_ref):
        out_ref[...] = jnp.maximum(
            aggp_ref[0, :n] + aggp_ref[1, :n] + b_ref[...], 0.0)

    return pl.pallas_call(
        body,
        out_shape=jax.ShapeDtypeStruct((n, dl), jnp.float32),
    )(aggp, b2)


def kernel(x, edge_index, edge_weight, W1, b1, W2, b2):
    n, _ = x.shape
    e = edge_weight.shape[0]
    dh = W1.shape[1]
    dl = W2.shape[1]
    npad = -(-n // (8 * NS)) * (8 * NS)

    src = edge_index[0]
    dst = edge_index[1]
    per_tile = -(-e // NW)
    et = -(-per_tile // EC) * EC
    epad = NW * et
    padn = epad - e
    src_p = jnp.concatenate(
        [src, jnp.zeros((padn,), src.dtype)]).reshape(NW, et)
    dst_p = jnp.concatenate(
        [dst, jnp.zeros((padn,), dst.dtype)]).reshape(NW, et)
    ew_p = jnp.concatenate(
        [edge_weight, jnp.zeros((padn,), edge_weight.dtype)]).reshape(NW, et)

    x_p = jnp.pad(x, ((0, npad - n), (0, 0)))
    degp = _deg_kernel(n, npad, et)(dst_p, ew_p)
    h1, self1, dinv = _prep_tc(x_p, W1, degp)
    norm2 = _norm_kernel(npad, et)(src_p, dst_p, ew_p, dinv)

    ch = et // EC
    src3 = src_p.reshape(NW, ch, EC)
    dst3 = dst_p.reshape(NW, ch, EC)
    norm3 = norm2.reshape(NW, ch, EC)

    zeros1 = jnp.zeros((npad, dh), jnp.float32)
    aggp1 = _agg_kernel(npad, dh, ch)(h1, src3, dst3, norm3, self1, zeros1)
    h2, self2 = _mid_tc(aggp1, b1, W2, dinv)

    zeros2 = jnp.zeros((npad, dl), jnp.float32)
    aggp2 = _agg_kernel(npad, dl, ch)(h2, src3, dst3, norm3, self2, zeros2)
    return _fin_tc(aggp2, b2, n)
